# Initial kernel scaffold; baseline (speedup 1.0000x reference)
import jax
import jax.numpy as jnp
from jax.experimental import pallas as pl


def kernel(inputs, indices, W, b):
    raise NotImplementedError("write your pallas kernel here")


# speedup vs baseline: 1.5416x; 1.5416x over previous
import functools

import jax
import jax.numpy as jnp
from jax import lax
from jax.experimental import pallas as pl
from jax.experimental.pallas import tpu as pltpu
from jax.experimental.pallas import tpu_sc as plsc

_NC = 2
_NS = 16
_NW = _NC * _NS
_LANES = 16


def _matmul(x, wr, bpat, block_m):
    n, c = x.shape
    d = wr.shape[1]

    def body(x_ref, w_ref, b_ref, o_ref):
        o_ref[...] = (
            jnp.dot(x_ref[...], w_ref[...], preferred_element_type=jnp.float32)
            + b_ref[...]
        )

    return pl.pallas_call(
        body,
        grid=(n // block_m,),
        in_specs=[
            pl.BlockSpec((block_m, c), lambda i: (i, 0)),
            pl.BlockSpec((c, d), lambda i: (0, 0)),
            pl.BlockSpec((1, d), lambda i: (0, 0)),
        ],
        out_specs=pl.BlockSpec((block_m, d), lambda i: (i, 0)),
        out_shape=jax.ShapeDtypeStruct((n, d), jnp.float32),
    )(x, wr, bpat)


def _sc_gather_sum(zf, idxt, np_, s, co):
    ch = np_ // _NW
    mesh = plsc.VectorSubcoreMesh(
        core_axis_name="cx", subcore_axis_name="sx", num_cores=_NC,
        num_subcores=_NS)

    @functools.partial(
        pl.kernel,
        out_type=jax.ShapeDtypeStruct((np_, co), jnp.float32),
        mesh=mesh,
        scratch_types=[
            pltpu.VMEM((ch,), jnp.int32),
            pltpu.VMEM((ch,), jnp.int32),
            pltpu.VMEM((ch, co), jnp.float32),
            pltpu.SemaphoreType.DMA,
        ],
    )
    def run(z_hbm, idxt_hbm, out_hbm, idx_v, gidx_v, acc_v, sem):
        wid = lax.axis_index("sx") * _NC + lax.axis_index("cx")
        base = pl.multiple_of(wid * ch, ch)

        def prep_gidx(sv):
            off = pl.multiple_of(sv * np_ + base, ch)
            pltpu.sync_copy(idxt_hbm.at[pl.ds(off, ch)], idx_v)

            def gbody(i, _):
                gidx_v[pl.ds(i * _LANES, _LANES)] = (
                    idx_v[pl.ds(i * _LANES, _LANES)] * s + sv)
                return 0

            lax.fori_loop(0, ch // _LANES, gbody, 0)

        prep_gidx(0)
        pltpu.async_copy(z_hbm.at[gidx_v], acc_v, sem).wait()

        def sbody(sv, _):
            prep_gidx(sv)
            pltpu.async_copy(z_hbm.at[gidx_v], acc_v, sem, add=True).wait()
            return 0

        lax.fori_loop(1, s, sbody, 0)

        pltpu.sync_copy(acc_v, out_hbm.at[pl.ds(base, ch)])

    return run(zf, idxt)


def kernel(inputs, indices, W, b):
    batch, n, c = inputs.shape
    n_nodes, s = indices.shape
    co = W.shape[1]

    x = inputs.reshape(n, c)
    wr = W.reshape(s, c, co).transpose(1, 0, 2).reshape(c, s * co)
    bpat = jnp.concatenate([b, jnp.zeros((s * co - co,), jnp.float32)])
    bpat = bpat.reshape(1, s * co)

    z = _matmul(x, wr, bpat, block_m=400)
    zf = z.reshape(n * s, co)

    ch = -(-n // _NW)
    ch = -(-ch // 8) * 8
    np_ = ch * _NW
    idx = indices.astype(jnp.int32)
    idxt = jnp.pad(idx, ((0, np_ - n), (0, 0))).T.reshape(-1)

    outp = _sc_gather_sum(zf, idxt, np_, s, co)
    return outp[:n].reshape(batch, n, co)


# speedup vs baseline: 2.0278x; 1.3154x over previous
import functools

import jax
import jax.numpy as jnp
from jax import lax
from jax.experimental import pallas as pl
from jax.experimental.pallas import tpu as pltpu
from jax.experimental.pallas import tpu_sc as plsc

_NC = 2
_NS = 16
_NW = _NC * _NS
_LANES = 16


def _matmul(x, wr, bpat, block_m):
    n, c = x.shape
    d = wr.shape[1]

    def body(x_ref, w_ref, b_ref, o_ref):
        o_ref[...] = (
            jnp.dot(x_ref[...], w_ref[...], preferred_element_type=jnp.float32)
            + b_ref[...]
        )

    return pl.pallas_call(
        body,
        grid=(n // block_m,),
        in_specs=[
            pl.BlockSpec((block_m, c), lambda i: (i, 0)),
            pl.BlockSpec((c, d), lambda i: (0, 0)),
            pl.BlockSpec((1, d), lambda i: (0, 0)),
        ],
        out_specs=pl.BlockSpec((block_m, d), lambda i: (i, 0)),
        out_shape=jax.ShapeDtypeStruct((n, d), jnp.float32),
    )(x, wr, bpat)


def _sc_gather_sum(zf, idxt3, np_, s, co):
    ch = np_ // _NW
    blk = s * ch
    mesh = plsc.VectorSubcoreMesh(
        core_axis_name="cx", subcore_axis_name="sx", num_cores=_NC,
        num_subcores=_NS)

    @functools.partial(
        pl.kernel,
        out_type=jax.ShapeDtypeStruct((np_, co), jnp.float32),
        mesh=mesh,
        scratch_types=[
            pltpu.VMEM((blk,), jnp.int32),
            pltpu.VMEM((blk,), jnp.int32),
            pltpu.VMEM((ch, co), jnp.float32),
            pltpu.SemaphoreType.DMA,
        ],
    )
    def run(z_hbm, idxt_hbm, out_hbm, idx_v, gidx_v, acc_v, sem):
        wid = lax.axis_index("sx") * _NC + lax.axis_index("cx")
        base = pl.multiple_of(wid * ch, ch)

        pltpu.sync_copy(idxt_hbm.at[pl.ds(pl.multiple_of(wid * blk, blk), blk)],
                        idx_v)

        def gouter(sv, _):
            def gbody(i, _):
                p = sv * ch + i * _LANES
                gidx_v[pl.ds(p, _LANES)] = idx_v[pl.ds(p, _LANES)] * s + sv
                return 0
            lax.fori_loop(0, ch // _LANES, gbody, 0, unroll=4)
            return 0

        lax.fori_loop(0, s, gouter, 0)

        pltpu.async_copy(z_hbm.at[gidx_v.at[pl.ds(0, ch)]], acc_v, sem).wait()

        def fire(sv, _):
            pltpu.async_copy(
                z_hbm.at[gidx_v.at[pl.ds(pl.multiple_of(sv * ch, ch), ch)]],
                acc_v, sem, add=True)
            return 0

        lax.fori_loop(1, s, fire, 0)

        def drain(sv, _):
            pltpu.make_async_copy(z_hbm.at[pl.ds(0, ch)], acc_v, sem).wait()
            return 0

        lax.fori_loop(1, s, drain, 0)

        pltpu.sync_copy(acc_v, out_hbm.at[pl.ds(base, ch)])

    return run(zf, idxt3)


def kernel(inputs, indices, W, b):
    batch, n, c = inputs.shape
    n_nodes, s = indices.shape
    co = W.shape[1]

    x = inputs.reshape(n, c)
    wr = W.reshape(s, c, co).transpose(1, 0, 2).reshape(c, s * co)
    bpat = jnp.concatenate([b, jnp.zeros((s * co - co,), jnp.float32)])
    bpat = bpat.reshape(1, s * co)

    z = _matmul(x, wr, bpat, block_m=400)
    zf = z.reshape(n * s, co)

    ch = -(-n // _NW)
    ch = -(-ch // 8) * 8
    np_ = ch * _NW
    idx = indices.astype(jnp.int32)
    idxt3 = (jnp.pad(idx, ((0, np_ - n), (0, 0)))
             .reshape(_NW, ch, s).transpose(0, 2, 1).reshape(-1))

    outp = _sc_gather_sum(zf, idxt3, np_, s, co)
    return outp[:n].reshape(batch, n, co)


# speedup vs baseline: 2.9637x; 1.4615x over previous
import functools

import jax
import jax.numpy as jnp
from jax import lax
from jax.experimental import pallas as pl
from jax.experimental.pallas import tpu as pltpu
from jax.experimental.pallas import tpu_sc as plsc

_NC = 2
_NS = 16
_NW = _NC * _NS
_LANES = 16


def _matmul(x, wr, b2, s):
    n, c = x.shape
    co = wr.shape[1] // s

    def body(x_ref, w_ref, b_ref, o_ref):
        z = jnp.dot(x_ref[...], w_ref[...], preferred_element_type=jnp.float32)

        @pl.when(pl.program_id(0) == 0)
        def _():
            o_ref[...] = (z + b_ref[...])[None]

        @pl.when(pl.program_id(0) != 0)
        def _():
            o_ref[...] = z[None]

    return pl.pallas_call(
        body,
        grid=(s,),
        in_specs=[
            pl.BlockSpec((n, c), lambda i: (0, 0)),
            pl.BlockSpec((c, co), lambda i: (0, i)),
            pl.BlockSpec((1, co), lambda i: (0, 0)),
        ],
        out_specs=pl.BlockSpec((1, n, co), lambda i: (i, 0, 0)),
        out_shape=jax.ShapeDtypeStruct((s, n, co), jnp.float32),
    )(x, wr, b2)


def _sc_gather_sum(zf, idxt3, np_, n, s, co):
    ch = np_ // _NW
    blk = s * ch
    mesh = plsc.VectorSubcoreMesh(
        core_axis_name="cx", subcore_axis_name="sx", num_cores=_NC,
        num_subcores=_NS)

    @functools.partial(
        pl.kernel,
        out_type=jax.ShapeDtypeStruct((np_, co), jnp.float32),
        mesh=mesh,
        scratch_types=[
            pltpu.VMEM((blk,), jnp.int32),
            pltpu.VMEM((blk,), jnp.int32),
            pltpu.VMEM((ch, co), jnp.float32),
            pltpu.SemaphoreType.DMA,
        ],
    )
    def run(z_hbm, idxt_hbm, out_hbm, idx_v, gidx_v, acc_v, sem):
        wid = lax.axis_index("sx") * _NC + lax.axis_index("cx")
        base = pl.multiple_of(wid * ch, ch)

        pltpu.sync_copy(idxt_hbm.at[pl.ds(pl.multiple_of(wid * blk, blk), blk)],
                        idx_v)

        def gouter(sv, _):
            def gbody(i, _):
                p = sv * ch + i * _LANES
                gidx_v[pl.ds(p, _LANES)] = idx_v[pl.ds(p, _LANES)] + sv * n
                return 0
            lax.fori_loop(0, ch // _LANES, gbody, 0, unroll=4)
            return 0

        lax.fori_loop(0, s, gouter, 0)

        pltpu.async_copy(z_hbm.at[gidx_v.at[pl.ds(0, ch)]], acc_v, sem).wait()

        def fire(sv, _):
            pltpu.async_copy(
                z_hbm.at[gidx_v.at[pl.ds(pl.multiple_of(sv * ch, ch), ch)]],
                acc_v, sem, add=True)
            return 0

        lax.fori_loop(1, s, fire, 0)

        def drain(sv, _):
            pltpu.make_async_copy(z_hbm.at[pl.ds(0, ch)], acc_v, sem).wait()
            return 0

        lax.fori_loop(1, s, drain, 0)

        pltpu.sync_copy(acc_v, out_hbm.at[pl.ds(base, ch)])

    return run(zf, idxt3)


def kernel(inputs, indices, W, b):
    batch, n, c = inputs.shape
    n_nodes, s = indices.shape
    co = W.shape[1]

    x = inputs.reshape(n, c)
    wr = W.reshape(s, c, co).transpose(1, 0, 2).reshape(c, s * co)

    z3 = _matmul(x, wr, b.reshape(1, co), s)
    zf = z3.reshape(s * n, co)

    ch = -(-n // _NW)
    ch = -(-ch // 8) * 8
    np_ = ch * _NW
    idx = indices.astype(jnp.int32)
    idxt3 = (jnp.pad(idx, ((0, np_ - n), (0, 0)))
             .reshape(_NW, ch, s).transpose(0, 2, 1).reshape(-1))

    outp = _sc_gather_sum(zf, idxt3, np_, n, s, co)
    return outp[:n].reshape(batch, n, co)


# speedup vs baseline: 4.0641x; 1.3713x over previous
import functools

import jax
import jax.numpy as jnp
from jax import lax
from jax.experimental import pallas as pl
from jax.experimental.pallas import tpu as pltpu
from jax.experimental.pallas import tpu_sc as plsc

_NC = 2
_NS = 16
_NW = _NC * _NS
_LANES = 16


def _matmul(x, wr, b2, s):
    n, c = x.shape
    co = wr.shape[1] // s

    def body(x_ref, w_ref, b_ref, o_ref):
        z = jnp.dot(x_ref[...], w_ref[...], preferred_element_type=jnp.float32)

        @pl.when(pl.program_id(0) == 0)
        def _():
            o_ref[...] = (z + b_ref[...])[None]

        @pl.when(pl.program_id(0) != 0)
        def _():
            o_ref[...] = z[None]

    return pl.pallas_call(
        body,
        grid=(s,),
        in_specs=[
            pl.BlockSpec((n, c), lambda i: (0, 0)),
            pl.BlockSpec((c, co), lambda i: (0, i)),
            pl.BlockSpec((1, co), lambda i: (0, 0)),
        ],
        out_specs=pl.BlockSpec((1, n, co), lambda i: (i, 0, 0)),
        out_shape=jax.ShapeDtypeStruct((s, n, co), jnp.float32),
    )(x, wr, b2)


def _sc_gather_sum(zf, idxt3, np_, n, s, co, ch0, ch1):
    mesh = plsc.VectorSubcoreMesh(
        core_axis_name="cx", subcore_axis_name="sx", num_cores=_NC,
        num_subcores=_NS)
    chmax = max(ch0, ch1)

    @functools.partial(
        pl.kernel,
        out_type=jax.ShapeDtypeStruct((np_, co), jnp.float32),
        mesh=mesh,
        scratch_types=[
            pltpu.VMEM((s * chmax,), jnp.int32),
            pltpu.VMEM((s * chmax,), jnp.int32),
            pltpu.VMEM((chmax, co), jnp.float32),
            pltpu.SemaphoreType.DMA,
        ],
    )
    def run(z_hbm, idxt_hbm, out_hbm, idx_v, gidx_v, acc_v, sem):
        cid = lax.axis_index("cx")
        sid = lax.axis_index("sx")

        def work(ch, blk_base, out_base):
            blk = s * ch
            pltpu.sync_copy(idxt_hbm.at[pl.ds(blk_base, blk)], idx_v.at[pl.ds(0, blk)])

            def gouter(sv, _):
                def gbody(i, _):
                    p = sv * ch + i * _LANES
                    gidx_v[pl.ds(p, _LANES)] = idx_v[pl.ds(p, _LANES)] + sv * n
                    return 0
                lax.fori_loop(0, ch // _LANES, gbody, 0, unroll=4)
                return 0

            lax.fori_loop(0, s, gouter, 0)

            acc = acc_v.at[pl.ds(0, ch)]
            pltpu.async_copy(z_hbm.at[gidx_v.at[pl.ds(0, ch)]], acc, sem).wait()

            def fire(sv, _):
                pltpu.async_copy(
                    z_hbm.at[gidx_v.at[pl.ds(pl.multiple_of(sv * ch, ch), ch)]],
                    acc, sem, add=True)
                return 0

            lax.fori_loop(1, s, fire, 0)

            def drain(sv, _):
                pltpu.make_async_copy(z_hbm.at[pl.ds(0, ch)], acc, sem).wait()
                return 0

            lax.fori_loop(1, s, drain, 0)

            pltpu.sync_copy(acc, out_hbm.at[pl.ds(out_base, ch)])

        @pl.when(cid == 0)
        def _():
            work(ch0, sid * (s * ch0), sid * ch0)

        @pl.when(cid == 1)
        def _():
            work(ch1, _NS * s * ch0 + sid * (s * ch1), _NS * ch0 + sid * ch1)

    return run(zf, idxt3)


def kernel(inputs, indices, W, b):
    batch, n, c = inputs.shape
    n_nodes, s = indices.shape
    co = W.shape[1]

    x = inputs.reshape(n, c)
    wr = W.reshape(s, c, co).transpose(1, 0, 2).reshape(c, s * co)

    z3 = _matmul(x, wr, b.reshape(1, co), s)
    zf = z3.reshape(s * n, co)

    ch0, ch1 = 512, 128
    np_ = _NS * (ch0 + ch1)
    idx = indices.astype(jnp.int32)
    idxp = jnp.pad(idx, ((0, np_ - n), (0, 0)))
    p0 = idxp[:_NS * ch0].reshape(_NS, ch0, s).transpose(0, 2, 1).reshape(-1)
    p1 = idxp[_NS * ch0:].reshape(_NS, ch1, s).transpose(0, 2, 1).reshape(-1)
    idxt3 = jnp.concatenate([p0, p1])

    outp = _sc_gather_sum(zf, idxt3, np_, n, s, co, ch0, ch1)
    return outp[:n].reshape(batch, n, co)
